# baseline (speedup 1.0000x reference)
import functools

import jax
import jax.numpy as jnp
from jax import lax
from jax.experimental import pallas as pl
from jax.experimental.pallas import tpu as pltpu
from jax.experimental.pallas import tpu_sc as plsc

B = 16384
IN_DIM = 784
HID = 400
LAT = 32
K = 8192

BB = 256
BBD = 512
KC = 512
EPB = K // (B // BB)

NC, NS = 2, 16
NW = NC * NS
B_PER_W = B // NW
CH = 128
NCH = B_PER_W // CH


def _enc_argmin_body(xt_ref, w1t_ref, b1_ref, w2t_ref, b2_ref, emb_ref,
                     zt_ref, idx_ref, emb_pad_ref,
                     eaug_ref, bestv_ref, bestk_ref):
    i = pl.program_id(0)
    emb_pad_ref[:, :LAT] = emb_ref[pl.ds(i * EPB, EPB), :]

    @pl.when(i == 0)
    def _():
        e = emb_ref[...]
        eaug_ref[:, :LAT] = e.astype(jnp.bfloat16)
        eaug_ref[:, LAT:LAT + 1] = jnp.sum(
            e * e, axis=1, keepdims=True).astype(jnp.bfloat16)

    xb = xt_ref[...].astype(jnp.bfloat16)
    ht = jnp.maximum(
        jnp.dot(w1t_ref[...], xb, preferred_element_type=jnp.float32)
        + b1_ref[...], 0.0)
    zt = (jnp.dot(w2t_ref[...], ht.astype(jnp.bfloat16),
                  preferred_element_type=jnp.float32)
          + b2_ref[...])
    zt_ref[...] = zt

    z_aug = jnp.concatenate(
        [-2.0 * zt, jnp.ones((1, BB), jnp.float32)],
        axis=0).astype(jnp.bfloat16)

    def scan_chunk(k, _):
        ea = eaug_ref[pl.ds(k * KC, KC), :]
        s = jnp.dot(ea, z_aug,
                    preferred_element_type=jnp.float32
                    ).astype(jnp.bfloat16)

        @pl.when(k == 0)
        def _():
            bestv_ref[...] = s
            bestk_ref[...] = jnp.zeros((KC, BB), jnp.bfloat16)

        @pl.when(k > 0)
        def _():
            bv = bestv_ref[...]
            upd = s < bv
            bestv_ref[...] = jnp.where(upd, s, bv)
            bestk_ref[...] = jnp.where(
                upd, k.astype(jnp.bfloat16), bestk_ref[...])

        return 0

    lax.fori_loop(0, K // KC, scan_chunk, 0)

    bv = bestv_ref[...]
    m = jnp.min(bv, axis=0, keepdims=True)
    jj = lax.broadcasted_iota(jnp.int32, (KC, BB), 0)
    gidx = bestk_ref[...].astype(jnp.int32) * KC + jj
    cand = jnp.where(bv == m, gidx, K)
    idx_ref[...] = jnp.min(cand, axis=0, keepdims=True)


def _decode_body(zq_ref, zt_ref, u1t_ref, c1_ref, u2t_ref, c2_ref,
                 outt_ref, loss_ref):
    zq = zq_ref[:, :LAT].astype(jnp.bfloat16)
    ii = lax.broadcasted_iota(jnp.int32, (LAT, LAT), 0)
    jj = lax.broadcasted_iota(jnp.int32, (LAT, LAT), 1)
    eye = (ii == jj).astype(jnp.bfloat16)
    zqt = lax.dot_general(eye, zq, (((1,), (1,)), ((), ())),
                          preferred_element_type=jnp.float32)

    d = zqt - zt_ref[...]
    partial = jnp.sum(d * d).reshape(1, 1)

    @pl.when(pl.program_id(0) == 0)
    def _():
        loss_ref[...] = jnp.zeros((1, 1), jnp.float32)

    loss_ref[...] += partial

    hdt = jnp.maximum(
        jnp.dot(u1t_ref[...], zqt.astype(jnp.bfloat16),
                preferred_element_type=jnp.float32)
        + c1_ref[...], 0.0)
    logits = (jnp.dot(u2t_ref[...], hdt.astype(jnp.bfloat16),
                      preferred_element_type=jnp.float32)
              + c2_ref[...])
    outt_ref[...] = 1.0 / (1.0 + jnp.exp(-logits))


def _sc_gather_body(emb_hbm, idx_hbm, out_hbm, idx_v, rows_v, sem):
    wid = lax.axis_index("s") * NC + lax.axis_index("c")
    base = wid * B_PER_W
    pltpu.sync_copy(idx_hbm.at[pl.ds(wid * NCH, NCH)], idx_v)
    copies = []
    for j in range(NCH):
        copies.append(pltpu.async_copy(
            emb_hbm.at[idx_v.at[j]], rows_v.at[pl.ds(j * CH, CH)], sem))
    for c in copies:
        c.wait()
    pltpu.sync_copy(rows_v, out_hbm.at[pl.ds(base, B_PER_W)])


@functools.cache
def _sc_gather():
    return functools.partial(
        pl.kernel,
        out_type=jax.ShapeDtypeStruct((B, 128), jnp.float32),
        mesh=plsc.VectorSubcoreMesh(core_axis_name="c", subcore_axis_name="s",
                                    num_cores=NC, num_subcores=NS),
        scratch_types=[
            pltpu.VMEM((NCH, CH), jnp.int32),
            pltpu.VMEM((B_PER_W, 128), jnp.float32),
            pltpu.SemaphoreType.DMA,
        ],
    )(_sc_gather_body)


def kernel(x, enc_w1, enc_b1, enc_w2, enc_b2, dec_w1, dec_b1, dec_w2, dec_b2,
           emb):
    grid = B // BB
    full = lambda shape: pl.BlockSpec(shape, lambda i: (0,) * len(shape))

    xt = x.T
    w1t = enc_w1.T.astype(jnp.bfloat16)
    w2t = enc_w2.T.astype(jnp.bfloat16)
    b1c = enc_b1.reshape(HID, 1)
    b2c = enc_b2.reshape(LAT, 1)

    zt, idx, emb_pad = pl.pallas_call(
        _enc_argmin_body,
        grid=(grid,),
        in_specs=[
            pl.BlockSpec((IN_DIM, BB), lambda i: (0, i)),
            full((HID, IN_DIM)),
            full((HID, 1)),
            full((LAT, HID)),
            full((LAT, 1)),
            full((K, LAT)),
        ],
        out_specs=[
            pl.BlockSpec((LAT, BB), lambda i: (0, i)),
            pl.BlockSpec((1, BB), lambda i: (0, i)),
            pl.BlockSpec((EPB, 128), lambda i: (i, 0)),
        ],
        out_shape=[
            jax.ShapeDtypeStruct((LAT, B), jnp.float32),
            jax.ShapeDtypeStruct((1, B), jnp.int32),
            jax.ShapeDtypeStruct((K, 128), jnp.float32),
        ],
        scratch_shapes=[
            pltpu.VMEM((K, LAT + 1), jnp.bfloat16),
            pltpu.VMEM((KC, BB), jnp.bfloat16),
            pltpu.VMEM((KC, BB), jnp.bfloat16),
        ],
    )(xt, w1t, b1c, w2t, b2c, emb)

    idx2d = idx.reshape(NW * NCH, CH)
    z_q = _sc_gather()(emb_pad, idx2d)

    u1t = dec_w1.T.astype(jnp.bfloat16)
    u2t = dec_w2.T.astype(jnp.bfloat16)
    c1c = dec_b1.reshape(HID, 1)
    c2c = dec_b2.reshape(IN_DIM, 1)

    outt, loss_sum = pl.pallas_call(
        _decode_body,
        grid=(B // BBD,),
        in_specs=[
            pl.BlockSpec((BBD, 128), lambda i: (i, 0)),
            pl.BlockSpec((LAT, BBD), lambda i: (0, i)),
            full((HID, LAT)),
            full((HID, 1)),
            full((IN_DIM, HID)),
            full((IN_DIM, 1)),
        ],
        out_specs=[
            pl.BlockSpec((IN_DIM, BBD), lambda i: (0, i)),
            pl.BlockSpec((1, 1), lambda i: (0, 0)),
        ],
        out_shape=[
            jax.ShapeDtypeStruct((IN_DIM, B), jnp.float32),
            jax.ShapeDtypeStruct((1, 1), jnp.float32),
        ],
    )(z_q, zt, u1t, c1c, u2t, c2c)

    x_recon = outt.T
    vq_loss = loss_sum[0, 0] * (1.25 / (B * LAT))
    return (x_recon, vq_loss)


# speedup vs baseline: 1.1027x; 1.1027x over previous
import functools

import jax
import jax.numpy as jnp
from jax import lax
from jax.experimental import pallas as pl
from jax.experimental.pallas import tpu as pltpu
from jax.experimental.pallas import tpu_sc as plsc

B = 16384
IN_DIM = 784
HID = 400
LAT = 32
K = 8192

BB = 256
BBD = 512
KC = 1024
EPB = K // (B // BB)

NC, NS = 2, 16
NW = NC * NS
B_PER_W = B // NW
CH = 128
NCH = B_PER_W // CH


def _enc_argmin_body(xt_ref, w1t_ref, b1_ref, w2t_ref, b2_ref, emb_ref,
                     zt_ref, idx_ref, emb_pad_ref,
                     eaug_ref, bestv_ref, bestk_ref):
    i = pl.program_id(0)
    emb_pad_ref[:, :LAT] = emb_ref[pl.ds(i * EPB, EPB), :]

    @pl.when(i == 0)
    def _():
        e = emb_ref[...]
        eaug_ref[:, :LAT] = e.astype(jnp.bfloat16)
        eaug_ref[:, LAT:LAT + 1] = jnp.sum(
            e * e, axis=1, keepdims=True).astype(jnp.bfloat16)

    xb = xt_ref[...].astype(jnp.bfloat16)
    ht = jnp.maximum(
        jnp.dot(w1t_ref[...], xb, preferred_element_type=jnp.float32)
        + b1_ref[...], 0.0)
    zt = (jnp.dot(w2t_ref[...], ht.astype(jnp.bfloat16),
                  preferred_element_type=jnp.float32)
          + b2_ref[...])
    zt_ref[...] = zt

    z_aug = jnp.concatenate(
        [-2.0 * zt, jnp.ones((1, BB), jnp.float32)],
        axis=0).astype(jnp.bfloat16)

    def scan_chunk(k, _):
        ea = eaug_ref[pl.ds(k * KC, KC), :]
        s = jnp.dot(ea, z_aug,
                    preferred_element_type=jnp.float32
                    ).astype(jnp.bfloat16)

        @pl.when(k == 0)
        def _():
            bestv_ref[...] = s
            bestk_ref[...] = jnp.zeros((KC, BB), jnp.bfloat16)

        @pl.when(k > 0)
        def _():
            bv = bestv_ref[...]
            upd = s < bv
            bestv_ref[...] = jnp.where(upd, s, bv)
            bestk_ref[...] = jnp.where(
                upd, k.astype(jnp.bfloat16), bestk_ref[...])

        return 0

    lax.fori_loop(0, K // KC, scan_chunk, 0)

    bv = bestv_ref[...]
    m = jnp.min(bv, axis=0, keepdims=True)
    jj = lax.broadcasted_iota(jnp.int32, (KC, BB), 0)
    gidx = bestk_ref[...].astype(jnp.int32) * KC + jj
    cand = jnp.where(bv == m, gidx, K)
    idx_ref[...] = jnp.min(cand, axis=0, keepdims=True)


def _decode_body(zq_ref, zt_ref, u1t_ref, c1_ref, u2t_ref, c2_ref,
                 outt_ref, loss_ref):
    zq = zq_ref[:, :LAT].astype(jnp.bfloat16)
    ii = lax.broadcasted_iota(jnp.int32, (LAT, LAT), 0)
    jj = lax.broadcasted_iota(jnp.int32, (LAT, LAT), 1)
    eye = (ii == jj).astype(jnp.bfloat16)
    zqt = lax.dot_general(eye, zq, (((1,), (1,)), ((), ())),
                          preferred_element_type=jnp.float32)

    d = zqt - zt_ref[...]
    partial = jnp.sum(d * d).reshape(1, 1)

    @pl.when(pl.program_id(0) == 0)
    def _():
        loss_ref[...] = jnp.zeros((1, 1), jnp.float32)

    loss_ref[...] += partial

    hdt = jnp.maximum(
        jnp.dot(u1t_ref[...], zqt.astype(jnp.bfloat16),
                preferred_element_type=jnp.float32)
        + c1_ref[...], 0.0)
    logits = (jnp.dot(u2t_ref[...], hdt.astype(jnp.bfloat16),
                      preferred_element_type=jnp.float32)
              + c2_ref[...])
    outt_ref[...] = 1.0 / (1.0 + jnp.exp(-logits))


def _sc_gather_body(emb_hbm, idx_hbm, out_hbm, idx_v, rows_v, sem):
    wid = lax.axis_index("s") * NC + lax.axis_index("c")
    base = wid * B_PER_W
    pltpu.sync_copy(idx_hbm.at[pl.ds(wid * NCH, NCH)], idx_v)
    copies = []
    for j in range(NCH):
        copies.append(pltpu.async_copy(
            emb_hbm.at[idx_v.at[j]], rows_v.at[pl.ds(j * CH, CH)], sem))
    for c in copies:
        c.wait()
    pltpu.sync_copy(rows_v, out_hbm.at[pl.ds(base, B_PER_W)])


@functools.cache
def _sc_gather():
    return functools.partial(
        pl.kernel,
        out_type=jax.ShapeDtypeStruct((B, 128), jnp.float32),
        mesh=plsc.VectorSubcoreMesh(core_axis_name="c", subcore_axis_name="s",
                                    num_cores=NC, num_subcores=NS),
        scratch_types=[
            pltpu.VMEM((NCH, CH), jnp.int32),
            pltpu.VMEM((B_PER_W, 128), jnp.float32),
            pltpu.SemaphoreType.DMA,
        ],
    )(_sc_gather_body)


def kernel(x, enc_w1, enc_b1, enc_w2, enc_b2, dec_w1, dec_b1, dec_w2, dec_b2,
           emb):
    grid = B // BB
    full = lambda shape: pl.BlockSpec(shape, lambda i: (0,) * len(shape))

    xt = x.T
    w1t = enc_w1.T.astype(jnp.bfloat16)
    w2t = enc_w2.T.astype(jnp.bfloat16)
    b1c = enc_b1.reshape(HID, 1)
    b2c = enc_b2.reshape(LAT, 1)

    zt, idx, emb_pad = pl.pallas_call(
        _enc_argmin_body,
        grid=(grid,),
        in_specs=[
            pl.BlockSpec((IN_DIM, BB), lambda i: (0, i)),
            full((HID, IN_DIM)),
            full((HID, 1)),
            full((LAT, HID)),
            full((LAT, 1)),
            full((K, LAT)),
        ],
        out_specs=[
            pl.BlockSpec((LAT, BB), lambda i: (0, i)),
            pl.BlockSpec((1, BB), lambda i: (0, i)),
            pl.BlockSpec((EPB, 128), lambda i: (i, 0)),
        ],
        out_shape=[
            jax.ShapeDtypeStruct((LAT, B), jnp.float32),
            jax.ShapeDtypeStruct((1, B), jnp.int32),
            jax.ShapeDtypeStruct((K, 128), jnp.float32),
        ],
        scratch_shapes=[
            pltpu.VMEM((K, LAT + 1), jnp.bfloat16),
            pltpu.VMEM((KC, BB), jnp.bfloat16),
            pltpu.VMEM((KC, BB), jnp.bfloat16),
        ],
    )(xt, w1t, b1c, w2t, b2c, emb)

    idx2d = idx.reshape(NW * NCH, CH)
    z_q = _sc_gather()(emb_pad, idx2d)

    u1t = dec_w1.T.astype(jnp.bfloat16)
    u2t = dec_w2.T.astype(jnp.bfloat16)
    c1c = dec_b1.reshape(HID, 1)
    c2c = dec_b2.reshape(IN_DIM, 1)

    outt, loss_sum = pl.pallas_call(
        _decode_body,
        grid=(B // BBD,),
        in_specs=[
            pl.BlockSpec((BBD, 128), lambda i: (i, 0)),
            pl.BlockSpec((LAT, BBD), lambda i: (0, i)),
            full((HID, LAT)),
            full((HID, 1)),
            full((IN_DIM, HID)),
            full((IN_DIM, 1)),
        ],
        out_specs=[
            pl.BlockSpec((IN_DIM, BBD), lambda i: (0, i)),
            pl.BlockSpec((1, 1), lambda i: (0, 0)),
        ],
        out_shape=[
            jax.ShapeDtypeStruct((IN_DIM, B), jnp.float32),
            jax.ShapeDtypeStruct((1, 1), jnp.float32),
        ],
    )(z_q, zt, u1t, c1c, u2t, c2c)

    x_recon = outt.T
    vq_loss = loss_sum[0, 0] * (1.25 / (B * LAT))
    return (x_recon, vq_loss)


# speedup vs baseline: 1.6268x; 1.4753x over previous
import functools

import jax
import jax.numpy as jnp
from jax import lax
from jax.experimental import pallas as pl
from jax.experimental.pallas import tpu as pltpu
from jax.experimental.pallas import tpu_sc as plsc

B = 16384
IN_DIM = 784
HID = 400
LAT = 32
K = 8192

BB = 256
BBD = 512
KC = 1024
EPB = K // (B // BB)

NC, NS = 2, 16
NW = NC * NS
B_PER_W = B // NW
CH = 128
NCH = B_PER_W // CH


def _enc_argmin_body(xt_ref, w1t_ref, b1_ref, w2t_ref, b2_ref, emb_ref,
                     zt_ref, idx_ref, emb_pad_ref,
                     eaug_ref, bestv_ref, bestk_ref):
    i = pl.program_id(0)
    emb_pad_ref[:, :LAT] = emb_ref[pl.ds(i * EPB, EPB), :]

    @pl.when(i == 0)
    def _():
        e = emb_ref[...]
        eaug_ref[:, :LAT] = e.astype(jnp.bfloat16)
        eaug_ref[:, LAT:LAT + 1] = jnp.sum(
            e * e, axis=1, keepdims=True).astype(jnp.bfloat16)

    xb = xt_ref[...].astype(jnp.bfloat16)
    ht = jnp.maximum(
        jnp.dot(w1t_ref[...], xb, preferred_element_type=jnp.float32)
        + b1_ref[...], 0.0)
    zt = (jnp.dot(w2t_ref[...], ht.astype(jnp.bfloat16),
                  preferred_element_type=jnp.float32)
          + b2_ref[...])
    zt_ref[...] = zt

    z_aug = jnp.concatenate(
        [-2.0 * zt, jnp.ones((1, BB), jnp.float32)],
        axis=0).astype(jnp.bfloat16)

    for k in range(K // KC):
        ea = eaug_ref[k * KC:(k + 1) * KC, :]
        s = jnp.dot(ea, z_aug,
                    preferred_element_type=jnp.float32
                    ).astype(jnp.bfloat16)
        if k == 0:
            bestv_ref[...] = s
            bestk_ref[...] = jnp.zeros((KC, BB), jnp.bfloat16)
        else:
            bv = bestv_ref[...]
            upd = s < bv
            bestv_ref[...] = jnp.where(upd, s, bv)
            bestk_ref[...] = jnp.where(
                upd, jnp.bfloat16(k), bestk_ref[...])

    bv = bestv_ref[...]
    m = jnp.min(bv, axis=0, keepdims=True)
    jj = lax.broadcasted_iota(jnp.int32, (KC, BB), 0)
    gidx = bestk_ref[...].astype(jnp.int32) * KC + jj
    cand = jnp.where(bv == m, gidx, K)
    idx_ref[...] = jnp.min(cand, axis=0, keepdims=True)


def _decode_body(zq_ref, zt_ref, u1t_ref, c1_ref, u2t_ref, c2_ref,
                 outt_ref, loss_ref):
    zq = zq_ref[:, :LAT].astype(jnp.bfloat16)
    ii = lax.broadcasted_iota(jnp.int32, (LAT, LAT), 0)
    jj = lax.broadcasted_iota(jnp.int32, (LAT, LAT), 1)
    eye = (ii == jj).astype(jnp.bfloat16)
    zqt = lax.dot_general(eye, zq, (((1,), (1,)), ((), ())),
                          preferred_element_type=jnp.float32)

    d = zqt - zt_ref[...]
    partial = jnp.sum(d * d).reshape(1, 1)

    @pl.when(pl.program_id(0) == 0)
    def _():
        loss_ref[...] = jnp.zeros((1, 1), jnp.float32)

    loss_ref[...] += partial

    hdt = jnp.maximum(
        jnp.dot(u1t_ref[...], zqt.astype(jnp.bfloat16),
                preferred_element_type=jnp.float32)
        + c1_ref[...], 0.0)
    logits = (jnp.dot(u2t_ref[...], hdt.astype(jnp.bfloat16),
                      preferred_element_type=jnp.float32)
              + c2_ref[...])
    outt_ref[...] = 1.0 / (1.0 + jnp.exp(-logits))


def _sc_gather_body(emb_hbm, idx_hbm, out_hbm, idx_v, rows_v, sem):
    wid = lax.axis_index("s") * NC + lax.axis_index("c")
    base = wid * B_PER_W
    pltpu.sync_copy(idx_hbm.at[pl.ds(wid * NCH, NCH)], idx_v)
    copies = []
    for j in range(NCH):
        copies.append(pltpu.async_copy(
            emb_hbm.at[idx_v.at[j]], rows_v.at[pl.ds(j * CH, CH)], sem))
    for c in copies:
        c.wait()
    pltpu.sync_copy(rows_v, out_hbm.at[pl.ds(base, B_PER_W)])


@functools.cache
def _sc_gather():
    return functools.partial(
        pl.kernel,
        out_type=jax.ShapeDtypeStruct((B, 128), jnp.float32),
        mesh=plsc.VectorSubcoreMesh(core_axis_name="c", subcore_axis_name="s",
                                    num_cores=NC, num_subcores=NS),
        scratch_types=[
            pltpu.VMEM((NCH, CH), jnp.int32),
            pltpu.VMEM((B_PER_W, 128), jnp.float32),
            pltpu.SemaphoreType.DMA,
        ],
    )(_sc_gather_body)


def kernel(x, enc_w1, enc_b1, enc_w2, enc_b2, dec_w1, dec_b1, dec_w2, dec_b2,
           emb):
    grid = B // BB
    full = lambda shape: pl.BlockSpec(shape, lambda i: (0,) * len(shape))

    xt = x.T
    w1t = enc_w1.T.astype(jnp.bfloat16)
    w2t = enc_w2.T.astype(jnp.bfloat16)
    b1c = enc_b1.reshape(HID, 1)
    b2c = enc_b2.reshape(LAT, 1)

    zt, idx, emb_pad = pl.pallas_call(
        _enc_argmin_body,
        grid=(grid,),
        in_specs=[
            pl.BlockSpec((IN_DIM, BB), lambda i: (0, i)),
            full((HID, IN_DIM)),
            full((HID, 1)),
            full((LAT, HID)),
            full((LAT, 1)),
            full((K, LAT)),
        ],
        out_specs=[
            pl.BlockSpec((LAT, BB), lambda i: (0, i)),
            pl.BlockSpec((1, BB), lambda i: (0, i)),
            pl.BlockSpec((EPB, 128), lambda i: (i, 0)),
        ],
        out_shape=[
            jax.ShapeDtypeStruct((LAT, B), jnp.float32),
            jax.ShapeDtypeStruct((1, B), jnp.int32),
            jax.ShapeDtypeStruct((K, 128), jnp.float32),
        ],
        scratch_shapes=[
            pltpu.VMEM((K, LAT + 1), jnp.bfloat16),
            pltpu.VMEM((KC, BB), jnp.bfloat16),
            pltpu.VMEM((KC, BB), jnp.bfloat16),
        ],
    )(xt, w1t, b1c, w2t, b2c, emb)

    idx2d = idx.reshape(NW * NCH, CH)
    z_q = _sc_gather()(emb_pad, idx2d)

    u1t = dec_w1.T.astype(jnp.bfloat16)
    u2t = dec_w2.T.astype(jnp.bfloat16)
    c1c = dec_b1.reshape(HID, 1)
    c2c = dec_b2.reshape(IN_DIM, 1)

    outt, loss_sum = pl.pallas_call(
        _decode_body,
        grid=(B // BBD,),
        in_specs=[
            pl.BlockSpec((BBD, 128), lambda i: (i, 0)),
            pl.BlockSpec((LAT, BBD), lambda i: (0, i)),
            full((HID, LAT)),
            full((HID, 1)),
            full((IN_DIM, HID)),
            full((IN_DIM, 1)),
        ],
        out_specs=[
            pl.BlockSpec((IN_DIM, BBD), lambda i: (0, i)),
            pl.BlockSpec((1, 1), lambda i: (0, 0)),
        ],
        out_shape=[
            jax.ShapeDtypeStruct((IN_DIM, B), jnp.float32),
            jax.ShapeDtypeStruct((1, 1), jnp.float32),
        ],
    )(z_q, zt, u1t, c1c, u2t, c2c)

    x_recon = outt.T
    vq_loss = loss_sum[0, 0] * (1.25 / (B * LAT))
    return (x_recon, vq_loss)


# speedup vs baseline: 1.9622x; 1.2062x over previous
import functools

import jax
import jax.numpy as jnp
from jax import lax
from jax.experimental import pallas as pl
from jax.experimental.pallas import tpu as pltpu
from jax.experimental.pallas import tpu_sc as plsc

B = 16384
IN_DIM = 784
HID = 400
LAT = 32
K = 8192

BB = 512
BBD = 512
KC = 512
EPB = K // (B // BB)

NC, NS = 2, 16
NW = NC * NS
B_PER_W = B // NW
CH = 128
NCH = B_PER_W // CH


def _enc_argmin_body(xt_ref, w1t_ref, b1_ref, w2t_ref, b2_ref, emb_ref,
                     zt_ref, idx_ref, emb_pad_ref,
                     eaug_ref, bestv_ref, bestk_ref):
    i = pl.program_id(0)
    emb_pad_ref[:, :LAT] = emb_ref[pl.ds(i * EPB, EPB), :]

    @pl.when(i == 0)
    def _():
        e = emb_ref[...]
        eaug_ref[:, :LAT] = e.astype(jnp.bfloat16)
        eaug_ref[:, LAT:LAT + 1] = jnp.sum(
            e * e, axis=1, keepdims=True).astype(jnp.bfloat16)

    xb = xt_ref[...].astype(jnp.bfloat16)
    ht = jnp.maximum(
        jnp.dot(w1t_ref[...], xb, preferred_element_type=jnp.float32)
        + b1_ref[...], 0.0)
    zt = (jnp.dot(w2t_ref[...], ht.astype(jnp.bfloat16),
                  preferred_element_type=jnp.float32)
          + b2_ref[...])
    zt_ref[...] = zt

    z_aug = jnp.concatenate(
        [-2.0 * zt, jnp.ones((1, BB), jnp.float32)],
        axis=0).astype(jnp.bfloat16)

    for k in range(K // KC):
        ea = eaug_ref[k * KC:(k + 1) * KC, :]
        s = jnp.dot(ea, z_aug,
                    preferred_element_type=jnp.float32
                    ).astype(jnp.bfloat16)
        if k == 0:
            bestv_ref[...] = s
            bestk_ref[...] = jnp.zeros((KC, BB), jnp.bfloat16)
        else:
            bv = bestv_ref[...]
            upd = s < bv
            bestv_ref[...] = jnp.where(upd, s, bv)
            bestk_ref[...] = jnp.where(
                upd, jnp.bfloat16(k), bestk_ref[...])

    bv = bestv_ref[...]
    m = jnp.min(bv, axis=0, keepdims=True)
    jj = lax.broadcasted_iota(jnp.int32, (KC, BB), 0)
    gidx = bestk_ref[...].astype(jnp.int32) * KC + jj
    cand = jnp.where(bv == m, gidx, K)
    idx_ref[...] = jnp.min(cand, axis=0, keepdims=True)


def _decode_body(zq_ref, zt_ref, u1t_ref, c1_ref, u2t_ref, c2_ref,
                 outt_ref, loss_ref):
    zq = zq_ref[:, :LAT].astype(jnp.bfloat16)
    ii = lax.broadcasted_iota(jnp.int32, (LAT, LAT), 0)
    jj = lax.broadcasted_iota(jnp.int32, (LAT, LAT), 1)
    eye = (ii == jj).astype(jnp.bfloat16)
    zqt = lax.dot_general(eye, zq, (((1,), (1,)), ((), ())),
                          preferred_element_type=jnp.float32)

    d = zqt - zt_ref[...]
    partial = jnp.sum(d * d).reshape(1, 1)

    @pl.when(pl.program_id(0) == 0)
    def _():
        loss_ref[...] = jnp.zeros((1, 1), jnp.float32)

    loss_ref[...] += partial

    hdt = jnp.maximum(
        jnp.dot(u1t_ref[...], zqt.astype(jnp.bfloat16),
                preferred_element_type=jnp.float32)
        + c1_ref[...], 0.0)
    logits = (jnp.dot(u2t_ref[...], hdt.astype(jnp.bfloat16),
                      preferred_element_type=jnp.float32)
              + c2_ref[...])
    outt_ref[...] = 1.0 / (1.0 + jnp.exp(-logits))


def _sc_gather_body(emb_hbm, idx_hbm, out_hbm, idx_v, rows_v, sem):
    wid = lax.axis_index("s") * NC + lax.axis_index("c")
    base = wid * B_PER_W
    pltpu.sync_copy(idx_hbm.at[pl.ds(wid * NCH, NCH)], idx_v)
    copies = []
    for j in range(NCH):
        copies.append(pltpu.async_copy(
            emb_hbm.at[idx_v.at[j]], rows_v.at[pl.ds(j * CH, CH)], sem))
    for c in copies:
        c.wait()
    pltpu.sync_copy(rows_v, out_hbm.at[pl.ds(base, B_PER_W)])


@functools.cache
def _sc_gather():
    return functools.partial(
        pl.kernel,
        out_type=jax.ShapeDtypeStruct((B, 128), jnp.float32),
        mesh=plsc.VectorSubcoreMesh(core_axis_name="c", subcore_axis_name="s",
                                    num_cores=NC, num_subcores=NS),
        scratch_types=[
            pltpu.VMEM((NCH, CH), jnp.int32),
            pltpu.VMEM((B_PER_W, 128), jnp.float32),
            pltpu.SemaphoreType.DMA,
        ],
    )(_sc_gather_body)


def kernel(x, enc_w1, enc_b1, enc_w2, enc_b2, dec_w1, dec_b1, dec_w2, dec_b2,
           emb):
    grid = B // BB
    full = lambda shape: pl.BlockSpec(shape, lambda i: (0,) * len(shape))

    xt = x.T
    w1t = enc_w1.T.astype(jnp.bfloat16)
    w2t = enc_w2.T.astype(jnp.bfloat16)
    b1c = enc_b1.reshape(HID, 1)
    b2c = enc_b2.reshape(LAT, 1)

    zt, idx, emb_pad = pl.pallas_call(
        _enc_argmin_body,
        grid=(grid,),
        in_specs=[
            pl.BlockSpec((IN_DIM, BB), lambda i: (0, i)),
            full((HID, IN_DIM)),
            full((HID, 1)),
            full((LAT, HID)),
            full((LAT, 1)),
            full((K, LAT)),
        ],
        out_specs=[
            pl.BlockSpec((LAT, BB), lambda i: (0, i)),
            pl.BlockSpec((1, BB), lambda i: (0, i)),
            pl.BlockSpec((EPB, 128), lambda i: (i, 0)),
        ],
        out_shape=[
            jax.ShapeDtypeStruct((LAT, B), jnp.float32),
            jax.ShapeDtypeStruct((1, B), jnp.int32),
            jax.ShapeDtypeStruct((K, 128), jnp.float32),
        ],
        scratch_shapes=[
            pltpu.VMEM((K, LAT + 1), jnp.bfloat16),
            pltpu.VMEM((KC, BB), jnp.bfloat16),
            pltpu.VMEM((KC, BB), jnp.bfloat16),
        ],
    )(xt, w1t, b1c, w2t, b2c, emb)

    idx2d = idx.reshape(NW * NCH, CH)
    z_q = _sc_gather()(emb_pad, idx2d)

    u1t = dec_w1.T.astype(jnp.bfloat16)
    u2t = dec_w2.T.astype(jnp.bfloat16)
    c1c = dec_b1.reshape(HID, 1)
    c2c = dec_b2.reshape(IN_DIM, 1)

    outt, loss_sum = pl.pallas_call(
        _decode_body,
        grid=(B // BBD,),
        in_specs=[
            pl.BlockSpec((BBD, 128), lambda i: (i, 0)),
            pl.BlockSpec((LAT, BBD), lambda i: (0, i)),
            full((HID, LAT)),
            full((HID, 1)),
            full((IN_DIM, HID)),
            full((IN_DIM, 1)),
        ],
        out_specs=[
            pl.BlockSpec((IN_DIM, BBD), lambda i: (0, i)),
            pl.BlockSpec((1, 1), lambda i: (0, 0)),
        ],
        out_shape=[
            jax.ShapeDtypeStruct((IN_DIM, B), jnp.float32),
            jax.ShapeDtypeStruct((1, 1), jnp.float32),
        ],
    )(z_q, zt, u1t, c1c, u2t, c2c)

    x_recon = outt.T
    vq_loss = loss_sum[0, 0] * (1.25 / (B * LAT))
    return (x_recon, vq_loss)


# speedup vs baseline: 2.4018x; 1.2240x over previous
import functools

import jax
import jax.numpy as jnp
from jax import lax
from jax.experimental import pallas as pl
from jax.experimental.pallas import tpu as pltpu
from jax.experimental.pallas import tpu_sc as plsc

B = 16384
IN_DIM = 784
HID = 400
LAT = 32
K = 8192

NH = 2
BH = B // NH
BB = 2048
BBD = 2048
KC = 256
GE = BH // BB
EPB = K // GE

NC, NS = 2, 16
NW = NC * NS
B_PER_W = BH // NW
CH = 128
NCH = B_PER_W // CH


def _enc_core(xt_ref, w1t_ref, b1_ref, w2t_ref, b2_ref, emb_ref,
              zt_ref, idx_ref, emb_pad_ref, eaug_ref, bestv_ref, bestk_ref):
    i = pl.program_id(0)
    if emb_pad_ref is not None:
        emb_pad_ref[:, :LAT] = emb_ref[pl.ds(i * EPB, EPB), :]

    @pl.when(i == 0)
    def _():
        e = emb_ref[...]
        eaug_ref[:, :LAT] = e.astype(jnp.bfloat16)
        eaug_ref[:, LAT:LAT + 1] = jnp.sum(
            e * e, axis=1, keepdims=True).astype(jnp.bfloat16)

    xb = xt_ref[...].astype(jnp.bfloat16)
    ht = jnp.maximum(
        jnp.dot(w1t_ref[...], xb, preferred_element_type=jnp.float32)
        + b1_ref[...], 0.0)
    zt = (jnp.dot(w2t_ref[...], ht.astype(jnp.bfloat16),
                  preferred_element_type=jnp.float32)
          + b2_ref[...])
    zt_ref[...] = zt

    z_aug = jnp.concatenate(
        [-2.0 * zt, jnp.ones((1, BB), jnp.float32)],
        axis=0).astype(jnp.bfloat16)

    for k in range(K // KC):
        ea = eaug_ref[k * KC:(k + 1) * KC, :]
        s = jnp.dot(ea, z_aug,
                    preferred_element_type=jnp.float32
                    ).astype(jnp.bfloat16)
        if k == 0:
            bestv_ref[...] = s
            bestk_ref[...] = jnp.zeros((KC, BB), jnp.bfloat16)
        else:
            bv = bestv_ref[...]
            upd = s < bv
            bestv_ref[...] = jnp.where(upd, s, bv)
            bestk_ref[...] = jnp.where(
                upd, jnp.bfloat16(k), bestk_ref[...])

    bv = bestv_ref[...]
    m = jnp.min(bv, axis=0, keepdims=True)
    jj = lax.broadcasted_iota(jnp.int32, (KC, BB), 0)
    gidx = bestk_ref[...].astype(jnp.int32) * KC + jj
    cand = jnp.where(bv == m, gidx, K)
    idx_ref[...] = jnp.min(cand, axis=0, keepdims=True)


def _enc_body_pad(xt_ref, w1t_ref, b1_ref, w2t_ref, b2_ref, emb_ref,
                  zt_ref, idx_ref, emb_pad_ref,
                  eaug_ref, bestv_ref, bestk_ref):
    _enc_core(xt_ref, w1t_ref, b1_ref, w2t_ref, b2_ref, emb_ref,
              zt_ref, idx_ref, emb_pad_ref, eaug_ref, bestv_ref, bestk_ref)


def _enc_body_nopad(xt_ref, w1t_ref, b1_ref, w2t_ref, b2_ref, emb_ref,
                    zt_ref, idx_ref,
                    eaug_ref, bestv_ref, bestk_ref):
    _enc_core(xt_ref, w1t_ref, b1_ref, w2t_ref, b2_ref, emb_ref,
              zt_ref, idx_ref, None, eaug_ref, bestv_ref, bestk_ref)


def _dec_core(zq_ref, zt_ref, u1t_ref, c1_ref, u2t_ref, c2_ref,
              outt_ref, loss_ref):
    zq = zq_ref[:, :LAT].astype(jnp.bfloat16)
    ii = lax.broadcasted_iota(jnp.int32, (LAT, LAT), 0)
    jj = lax.broadcasted_iota(jnp.int32, (LAT, LAT), 1)
    eye = (ii == jj).astype(jnp.bfloat16)
    zqt = lax.dot_general(eye, zq, (((1,), (1,)), ((), ())),
                          preferred_element_type=jnp.float32)

    d = zqt - zt_ref[...]
    partial = jnp.sum(d * d).reshape(1, 1)

    @pl.when(pl.program_id(0) == 0)
    def _():
        loss_ref[...] = jnp.zeros((1, 1), jnp.float32)

    loss_ref[...] += partial

    hdt = jnp.maximum(
        jnp.dot(u1t_ref[...], zqt.astype(jnp.bfloat16),
                preferred_element_type=jnp.float32)
        + c1_ref[...], 0.0)
    logits = (jnp.dot(u2t_ref[...], hdt.astype(jnp.bfloat16),
                      preferred_element_type=jnp.float32)
              + c2_ref[...])
    outt_ref[...] = 1.0 / (1.0 + jnp.exp(-logits))


def _dec_body0(zq_ref, zt_ref, u1t_ref, c1_ref, u2t_ref, c2_ref,
               outt_ref, loss_ref):
    _dec_core(zq_ref, zt_ref, u1t_ref, c1_ref, u2t_ref, c2_ref,
              outt_ref, loss_ref)


def _dec_body1(prev_ref, zq_ref, zt_ref, u1t_ref, c1_ref, u2t_ref, c2_ref,
               outt_ref, loss_ref):
    del prev_ref
    _dec_core(zq_ref, zt_ref, u1t_ref, c1_ref, u2t_ref, c2_ref,
              outt_ref, loss_ref)


def _sc_gather_body(emb_hbm, idx_hbm, out_hbm, idx_v, rows_v, gsem, wsem):
    wid = lax.axis_index("s") * NC + lax.axis_index("c")
    base = wid * B_PER_W
    pltpu.sync_copy(idx_hbm.at[pl.ds(wid * NCH, NCH)], idx_v)
    gathers = []
    for j in range(NCH):
        gathers.append(pltpu.async_copy(
            emb_hbm.at[idx_v.at[j]], rows_v.at[pl.ds(j * CH, CH)], gsem))
    writes = []
    for j in range(NCH):
        gathers[j].wait()
        writes.append(pltpu.async_copy(
            rows_v.at[pl.ds(j * CH, CH)],
            out_hbm.at[pl.ds(base + j * CH, CH)], wsem))
    for w in writes:
        w.wait()


@functools.cache
def _sc_gather():
    return functools.partial(
        pl.kernel,
        out_type=jax.ShapeDtypeStruct((BH, 128), jnp.float32),
        mesh=plsc.VectorSubcoreMesh(core_axis_name="c", subcore_axis_name="s",
                                    num_cores=NC, num_subcores=NS),
        scratch_types=[
            pltpu.VMEM((NCH, CH), jnp.int32),
            pltpu.VMEM((B_PER_W, 128), jnp.float32),
            pltpu.SemaphoreType.DMA,
            pltpu.SemaphoreType.DMA,
        ],
    )(_sc_gather_body)


def _enc_call(xt, w1t, b1c, w2t, b2c, emb, half):
    full = lambda shape: pl.BlockSpec(shape, lambda i: (0,) * len(shape))
    off = half * GE
    in_specs = [
        pl.BlockSpec((IN_DIM, BB), lambda i: (0, i + off)),
        full((HID, IN_DIM)),
        full((HID, 1)),
        full((LAT, HID)),
        full((LAT, 1)),
        full((K, LAT)),
    ]
    out_specs = [
        pl.BlockSpec((LAT, BB), lambda i: (0, i)),
        pl.BlockSpec((1, BB), lambda i: (0, i)),
    ]
    out_shape = [
        jax.ShapeDtypeStruct((LAT, BH), jnp.float32),
        jax.ShapeDtypeStruct((1, BH), jnp.int32),
    ]
    body = _enc_body_nopad
    if half == 0:
        out_specs.append(pl.BlockSpec((EPB, 128), lambda i: (i, 0)))
        out_shape.append(jax.ShapeDtypeStruct((K, 128), jnp.float32))
        body = _enc_body_pad
    return pl.pallas_call(
        body,
        grid=(GE,),
        in_specs=in_specs,
        out_specs=out_specs,
        out_shape=out_shape,
        scratch_shapes=[
            pltpu.VMEM((K, LAT + 1), jnp.bfloat16),
            pltpu.VMEM((KC, BB), jnp.bfloat16),
            pltpu.VMEM((KC, BB), jnp.bfloat16),
        ],
    )(xt, w1t, b1c, w2t, b2c, emb)


def _dec_call(z_q, zt, u1t, c1c, u2t, c2c, half, prev=None):
    full = lambda shape: pl.BlockSpec(shape, lambda i: (0,) * len(shape))
    off = half * (BH // BBD)
    in_specs = [
        pl.BlockSpec((BBD, 128), lambda i: (i, 0)),
        pl.BlockSpec((LAT, BBD), lambda i: (0, i)),
        full((HID, LAT)),
        full((HID, 1)),
        full((IN_DIM, HID)),
        full((IN_DIM, 1)),
    ]
    args = [z_q, zt, u1t, c1c, u2t, c2c]
    body = _dec_body0
    aliases = {}
    if prev is not None:
        in_specs.insert(0, pl.BlockSpec(memory_space=pl.ANY))
        args.insert(0, prev)
        body = _dec_body1
        aliases = {0: 0}
    return pl.pallas_call(
        body,
        grid=(BH // BBD,),
        in_specs=in_specs,
        out_specs=[
            pl.BlockSpec((IN_DIM, BBD), lambda i: (0, i + off)),
            pl.BlockSpec((1, 1), lambda i: (0, 0)),
        ],
        out_shape=[
            jax.ShapeDtypeStruct((IN_DIM, B), jnp.float32),
            jax.ShapeDtypeStruct((1, 1), jnp.float32),
        ],
        input_output_aliases=aliases,
    )(*args)


def kernel(x, enc_w1, enc_b1, enc_w2, enc_b2, dec_w1, dec_b1, dec_w2, dec_b2,
           emb):
    xt = x.T
    w1t = enc_w1.T.astype(jnp.bfloat16)
    w2t = enc_w2.T.astype(jnp.bfloat16)
    b1c = enc_b1.reshape(HID, 1)
    b2c = enc_b2.reshape(LAT, 1)
    u1t = dec_w1.T.astype(jnp.bfloat16)
    u2t = dec_w2.T.astype(jnp.bfloat16)
    c1c = dec_b1.reshape(HID, 1)
    c2c = dec_b2.reshape(IN_DIM, 1)

    zt0, idx0, emb_pad = _enc_call(xt, w1t, b1c, w2t, b2c, emb, 0)
    z_q0 = _sc_gather()(emb_pad, idx0.reshape(NW * NCH, CH))

    zt1, idx1 = _enc_call(xt, w1t, b1c, w2t, b2c, emb, 1)
    z_q1 = _sc_gather()(emb_pad, idx1.reshape(NW * NCH, CH))

    out_half, loss0 = _dec_call(z_q0, zt0, u1t, c1c, u2t, c2c, 0)
    outt, loss1 = _dec_call(z_q1, zt1, u1t, c1c, u2t, c2c, 1, prev=out_half)

    x_recon = outt.T
    vq_loss = (loss0[0, 0] + loss1[0, 0]) * (1.25 / (B * LAT))
    return (x_recon, vq_loss)
